# baseline (speedup 1.0000x reference)
import functools

import jax
import jax.numpy as jnp
from jax import lax
from jax.experimental import pallas as pl
from jax.experimental.pallas import tpu as pltpu
from jax.experimental.pallas import tpu_sc as plsc

N = 10000
D = 128
H = 128
E = 320000
D2 = 144

BN = 1000
NW = 32
CA = 128
CHUNKS_A = E // CA
FULL_A = CHUNKS_A // NW
EXTRA_A = CHUNKS_A - FULL_A * NW
CB = 128
CHUNKS_B = E // CB
FULL_B = CHUNKS_B // NW
EXTRA_B = CHUNKS_B - FULL_B * NW
RPS = 624
TAIL = N - 16 * RPS
TAIL0 = 16 * RPS


def _tc1_body(x_ref, wn_ref, ws_ref, b1_ref, y_ref, xs_ref):
    x = x_ref[...]
    yv = jnp.dot(x, wn_ref[...], preferred_element_type=jnp.float32)
    y_ref[...] = jnp.concatenate([yv, jnp.ones((BN, 16), jnp.float32)], axis=1)
    xs_ref[...] = (
        jnp.dot(x, ws_ref[...], preferred_element_type=jnp.float32) + b1_ref[...]
    )


def _tc1(x, W_neigh1, W_self1, b1):
    return pl.pallas_call(
        _tc1_body,
        grid=(N // BN,),
        in_specs=[
            pl.BlockSpec((BN, D), lambda i: (i, 0)),
            pl.BlockSpec((D, H), lambda i: (0, 0)),
            pl.BlockSpec((D, H), lambda i: (0, 0)),
            pl.BlockSpec((1, H), lambda i: (0, 0)),
        ],
        out_specs=[
            pl.BlockSpec((BN, D2), lambda i: (i, 0)),
            pl.BlockSpec((BN, H), lambda i: (i, 0)),
        ],
        out_shape=[
            jax.ShapeDtypeStruct((N, D2), jnp.float32),
            jax.ShapeDtypeStruct((N, H), jnp.float32),
        ],
    )(x, W_neigh1, W_self1, b1[None, :])


def _tc2_body(xs_ref, f0_ref, f1_ref, w2_ref, b2_ref, zs_ref):
    f = f0_ref[...] + f1_ref[...]
    agg = f[:, :D]
    deg = jnp.maximum(f[:, D:D + 1], 1.0)
    h = jnp.maximum(xs_ref[...] + agg / deg, 0.0)
    pair = jnp.dot(h, w2_ref[...], preferred_element_type=jnp.float32) + b2_ref[...]
    cols = lax.broadcasted_iota(jnp.int32, (BN, 16), 1)
    z = pair[:, 0:1]
    s = pair[:, 1:2]
    zs_ref[...] = jnp.where(
        cols == 0, z, jnp.where(cols == 1, s, jnp.where(cols == 2, deg, 0.0))
    )


def _tc2(xs, f0, f1, W2, b2v):
    return pl.pallas_call(
        _tc2_body,
        grid=(N // BN,),
        in_specs=[
            pl.BlockSpec((BN, H), lambda i: (i, 0)),
            pl.BlockSpec((BN, D2), lambda i: (i, 0)),
            pl.BlockSpec((BN, D2), lambda i: (i, 0)),
            pl.BlockSpec((H, 2), lambda i: (0, 0)),
            pl.BlockSpec((1, 2), lambda i: (0, 0)),
        ],
        out_specs=pl.BlockSpec((BN, 16), lambda i: (i, 0)),
        out_shape=jax.ShapeDtypeStruct((N, 16), jnp.float32),
    )(xs, f0, f1, W2, b2v)


def _tc3_body(zs_ref, a0_ref, a1_ref, o_ref):
    zagg = a0_ref[:, 0:1] + a1_ref[:, 0:1]
    s = zs_ref[:, 1:2]
    deg = zs_ref[:, 2:3]
    o_ref[...] = jax.nn.sigmoid(s + zagg / deg)


def _tc3(zs, a0, a1):
    return pl.pallas_call(
        _tc3_body,
        grid=(N // BN,),
        in_specs=[
            pl.BlockSpec((BN, 16), lambda i: (i, 0)),
            pl.BlockSpec((BN, 16), lambda i: (i, 0)),
            pl.BlockSpec((BN, 16), lambda i: (i, 0)),
        ],
        out_specs=pl.BlockSpec((BN, 1), lambda i: (i, 0)),
        out_shape=jax.ShapeDtypeStruct((N, 1), jnp.float32),
    )(zs, a0, a1)


def _sc_agg_wide(y, src2d, dst2d, zeros144):
    mesh = plsc.VectorSubcoreMesh(core_axis_name="c", subcore_axis_name="s")
    NBUF = 2
    K = NBUF - 1
    GROUPS = FULL_A // NBUF

    @functools.partial(
        pl.kernel,
        out_type=jax.ShapeDtypeStruct((2, N, D2), jnp.float32),
        mesh=mesh,
        scratch_types=[
            [pltpu.VMEM((CA,), jnp.int32)] * NBUF,
            [pltpu.VMEM((CA,), jnp.int32)] * NBUF,
            [pltpu.VMEM((CA, D2), jnp.float32)] * NBUF,
            pltpu.VMEM_SHARED((N, D2), jnp.float32),
            [pltpu.SemaphoreType.DMA] * NBUF,
            [pltpu.SemaphoreType.DMA] * NBUF,
            [pltpu.SemaphoreType.DMA] * NBUF,
        ],
        compiler_params=pltpu.CompilerParams(use_tc_tiling_on_sc=False),
    )
    def k(y_hbm, src_hbm, dst_hbm, z144_hbm, feat_hbm,
          srcs, dsts, rows, acc_y, isem, gsem, ssem):
        c = lax.axis_index("c")
        s = lax.axis_index("s")
        w = s * 2 + c
        r0 = pl.multiple_of(s * RPS, 8)

        pltpu.sync_copy(z144_hbm.at[pl.ds(r0, RPS)], acc_y.at[pl.ds(r0, RPS)])

        @pl.when(s == 15)
        def _():
            pltpu.sync_copy(z144_hbm.at[pl.ds(TAIL0, TAIL)],
                            acc_y.at[pl.ds(TAIL0, TAIL)])

        plsc.subcore_barrier()

        c0 = w * FULL_A

        for j in range(K):
            pltpu.sync_copy(src_hbm.at[c0 + j], srcs[j])
            pltpu.sync_copy(dst_hbm.at[c0 + j], dsts[j])
            pltpu.async_copy(y_hbm.at[srcs[j]], rows[j], gsem[j])
        pltpu.async_copy(src_hbm.at[c0 + K], srcs[K], isem[K])
        pltpu.async_copy(dst_hbm.at[c0 + K], dsts[K], isem[K])

        def group(g, carry):
            for b in range(NBUF):
                i = g * NBUF + b
                bj = (b + K) % NBUF

                @pl.when(i + K < FULL_A)
                def _():
                    pltpu.make_async_copy(src_hbm.at[0], srcs[bj],
                                          isem[bj]).wait()
                    pltpu.make_async_copy(dst_hbm.at[0], dsts[bj],
                                          isem[bj]).wait()
                    pltpu.async_copy(y_hbm.at[srcs[bj]], rows[bj], gsem[bj])

                pltpu.make_async_copy(y_hbm.at[srcs[b]], rows[b],
                                      gsem[b]).wait()
                pltpu.async_copy(rows[b], acc_y.at[dsts[b]], ssem[b],
                                 add=True)
                pltpu.make_async_copy(rows[b], acc_y.at[dsts[b]],
                                      ssem[b]).wait()

                @pl.when(i + NBUF < FULL_A)
                def _():
                    pltpu.async_copy(src_hbm.at[c0 + i + NBUF], srcs[b],
                                     isem[b])
                    pltpu.async_copy(dst_hbm.at[c0 + i + NBUF], dsts[b],
                                     isem[b])
            return carry

        lax.fori_loop(0, GROUPS, group, 0)

        @pl.when(w < EXTRA_A)
        def _():
            xc = NW * FULL_A + w
            pltpu.sync_copy(src_hbm.at[xc], srcs[0])
            pltpu.sync_copy(dst_hbm.at[xc], dsts[0])
            pltpu.async_copy(y_hbm.at[srcs[0]], rows[0], gsem[0]).wait()
            pltpu.sync_copy(rows[0], acc_y.at[dsts[0]], add=True)

        plsc.subcore_barrier()
        pltpu.sync_copy(acc_y.at[pl.ds(r0, RPS)], feat_hbm.at[c, pl.ds(r0, RPS)])

        @pl.when(s == 15)
        def _():
            pltpu.sync_copy(acc_y.at[pl.ds(TAIL0, TAIL)],
                            feat_hbm.at[c, pl.ds(TAIL0, TAIL)])

    return k(y, src2d, dst2d, zeros144)


def _sc_agg_narrow(zs, src2d, dst2d, zeros16):
    mesh = plsc.VectorSubcoreMesh(core_axis_name="c", subcore_axis_name="s")
    NBUF = 6
    GROUPS = FULL_B // NBUF

    @functools.partial(
        pl.kernel,
        out_type=jax.ShapeDtypeStruct((2, N, 16), jnp.float32),
        mesh=mesh,
        scratch_types=[
            pltpu.VMEM((FULL_B, CB), jnp.int32),
            pltpu.VMEM((FULL_B, CB), jnp.int32),
            pltpu.VMEM((CB,), jnp.int32),
            pltpu.VMEM((CB,), jnp.int32),
            [pltpu.VMEM((CB, 16), jnp.float32)] * NBUF,
            pltpu.VMEM_SHARED((N, 16), jnp.float32),
            [pltpu.SemaphoreType.DMA] * NBUF,
            [pltpu.SemaphoreType.DMA] * NBUF,
            pltpu.SemaphoreType.DMA,
        ],
        compiler_params=pltpu.CompilerParams(use_tc_tiling_on_sc=False),
    )
    def k(zs_hbm, src_hbm, dst_hbm, z16_hbm, out_hbm,
          src_v, dst_v, srcx_v, dstx_v, rows, acc, gsem, ssem, sem):
        c = lax.axis_index("c")
        s = lax.axis_index("s")
        w = s * 2 + c
        r0 = pl.multiple_of(s * RPS, 8)

        pltpu.sync_copy(z16_hbm.at[pl.ds(r0, RPS)], acc.at[pl.ds(r0, RPS)])

        @pl.when(s == 15)
        def _():
            pltpu.sync_copy(z16_hbm.at[pl.ds(TAIL0, TAIL)],
                            acc.at[pl.ds(TAIL0, TAIL)])

        c0 = w * FULL_B
        pltpu.sync_copy(src_hbm.at[pl.ds(c0, FULL_B)], src_v)
        pltpu.sync_copy(dst_hbm.at[pl.ds(c0, FULL_B)], dst_v)
        plsc.subcore_barrier()

        for b in range(NBUF):
            pltpu.async_copy(zs_hbm.at[src_v.at[b]], rows[b], gsem[b])

        def group(g, carry):
            for b in range(NBUF):
                i = g * NBUF + b
                pltpu.make_async_copy(zs_hbm.at[src_v.at[b]], rows[b],
                                      gsem[b]).wait()
                pltpu.async_copy(rows[b], acc.at[dst_v.at[i]], ssem[b],
                                 add=True)
                pltpu.make_async_copy(rows[b], acc.at[dst_v.at[i]],
                                      ssem[b]).wait()

                @pl.when(i + NBUF < FULL_B)
                def _():
                    pltpu.async_copy(zs_hbm.at[src_v.at[i + NBUF]], rows[b],
                                     gsem[b])
            return carry

        lax.fori_loop(0, GROUPS, group, 0)

        @pl.when(w < EXTRA_B)
        def _():
            xc = NW * FULL_B + w
            pltpu.sync_copy(src_hbm.at[xc], srcx_v)
            pltpu.sync_copy(dst_hbm.at[xc], dstx_v)
            pltpu.async_copy(zs_hbm.at[srcx_v], rows[0], sem).wait()
            pltpu.sync_copy(rows[0], acc.at[dstx_v], add=True)

        plsc.subcore_barrier()
        pltpu.sync_copy(acc.at[pl.ds(r0, RPS)], out_hbm.at[c, pl.ds(r0, RPS)])

        @pl.when(s == 15)
        def _():
            pltpu.sync_copy(acc.at[pl.ds(TAIL0, TAIL)],
                            out_hbm.at[c, pl.ds(TAIL0, TAIL)])

    return k(zs, src2d, dst2d, zeros16)


def kernel(x, edge_index, W_self1, W_neigh1, b1, W_self2, W_neigh2, b2):
    srcA = edge_index[0].reshape(CHUNKS_A, CA)
    dstA = edge_index[1].reshape(CHUNKS_A, CA)
    srcB = edge_index[0].reshape(CHUNKS_B, CB)
    dstB = edge_index[1].reshape(CHUNKS_B, CB)

    y, xs = _tc1(x, W_neigh1, W_self1, b1)

    zeros144 = jnp.zeros((N, D2), jnp.float32)
    zeros16 = jnp.zeros((N, 16), jnp.float32)

    feat = _sc_agg_wide(y, srcA, dstA, zeros144)

    W2 = jnp.concatenate([W_neigh2, W_self2], axis=1)
    b2v = jnp.concatenate([jnp.zeros((1,), jnp.float32), b2])[None, :]
    zs = _tc2(xs, feat[0], feat[1], W2, b2v)

    accB = _sc_agg_narrow(zs, srcB, dstB, zeros16)

    return _tc3(zs, accB[0], accB[1])


# speedup vs baseline: 1.0826x; 1.0826x over previous
import functools

import jax
import jax.numpy as jnp
from jax import lax
from jax.experimental import pallas as pl
from jax.experimental.pallas import tpu as pltpu
from jax.experimental.pallas import tpu_sc as plsc

N = 10000
D = 128
H = 128
E = 320000

BN = 1000
NW = 32
CA = 128
CHUNKS_A = E // CA
FULL_A = CHUNKS_A // NW
EXTRA_A = CHUNKS_A - FULL_A * NW
CB = 128
CHUNKS_B = E // CB
FULL_B = CHUNKS_B // NW
EXTRA_B = CHUNKS_B - FULL_B * NW
RPS = 624
TAIL = N - 16 * RPS
TAIL0 = 16 * RPS


def _tc1_body(x_ref, wn_ref, ws_ref, b1_ref, y_ref, xs_ref):
    x = x_ref[...]
    y_ref[...] = jnp.dot(x, wn_ref[...], preferred_element_type=jnp.float32)
    xs_ref[...] = (
        jnp.dot(x, ws_ref[...], preferred_element_type=jnp.float32) + b1_ref[...]
    )


def _tc1(x, W_neigh1, W_self1, b1):
    return pl.pallas_call(
        _tc1_body,
        grid=(N // BN,),
        in_specs=[
            pl.BlockSpec((BN, D), lambda i: (i, 0)),
            pl.BlockSpec((D, H), lambda i: (0, 0)),
            pl.BlockSpec((D, H), lambda i: (0, 0)),
            pl.BlockSpec((1, H), lambda i: (0, 0)),
        ],
        out_specs=[
            pl.BlockSpec((BN, H), lambda i: (i, 0)),
            pl.BlockSpec((BN, H), lambda i: (i, 0)),
        ],
        out_shape=[
            jax.ShapeDtypeStruct((N, H), jnp.float32),
            jax.ShapeDtypeStruct((N, H), jnp.float32),
        ],
    )(x, W_neigh1, W_self1, b1[None, :])


def _tc2_body(xs_ref, f0_ref, f1_ref, d0_ref, d1_ref, w2_ref, b2_ref, zs_ref):
    agg = f0_ref[...] + f1_ref[...]
    deg = jnp.maximum(d0_ref[:, 0:1] + d1_ref[:, 0:1], 1.0)
    h = jnp.maximum(xs_ref[...] + agg / deg, 0.0)
    pair = jnp.dot(h, w2_ref[...], preferred_element_type=jnp.float32) + b2_ref[...]
    cols = lax.broadcasted_iota(jnp.int32, (BN, 16), 1)
    z = pair[:, 0:1]
    s = pair[:, 1:2]
    zs_ref[...] = jnp.where(
        cols == 0, z, jnp.where(cols == 1, s, jnp.where(cols == 2, deg, 0.0))
    )


def _tc2(xs, f0, f1, d0, d1, W2, b2v):
    return pl.pallas_call(
        _tc2_body,
        grid=(N // BN,),
        in_specs=[
            pl.BlockSpec((BN, H), lambda i: (i, 0)),
            pl.BlockSpec((BN, H), lambda i: (i, 0)),
            pl.BlockSpec((BN, H), lambda i: (i, 0)),
            pl.BlockSpec((BN, 16), lambda i: (i, 0)),
            pl.BlockSpec((BN, 16), lambda i: (i, 0)),
            pl.BlockSpec((H, 2), lambda i: (0, 0)),
            pl.BlockSpec((1, 2), lambda i: (0, 0)),
        ],
        out_specs=pl.BlockSpec((BN, 16), lambda i: (i, 0)),
        out_shape=jax.ShapeDtypeStruct((N, 16), jnp.float32),
    )(xs, f0, f1, d0, d1, W2, b2v)


def _tc3_body(zs_ref, a0_ref, a1_ref, o_ref):
    zagg = a0_ref[:, 0:1] + a1_ref[:, 0:1]
    s = zs_ref[:, 1:2]
    deg = zs_ref[:, 2:3]
    o_ref[...] = jax.nn.sigmoid(s + zagg / deg)


def _tc3(zs, a0, a1):
    return pl.pallas_call(
        _tc3_body,
        grid=(N // BN,),
        in_specs=[
            pl.BlockSpec((BN, 16), lambda i: (i, 0)),
            pl.BlockSpec((BN, 16), lambda i: (i, 0)),
            pl.BlockSpec((BN, 16), lambda i: (i, 0)),
        ],
        out_specs=pl.BlockSpec((BN, 1), lambda i: (i, 0)),
        out_shape=jax.ShapeDtypeStruct((N, 1), jnp.float32),
    )(zs, a0, a1)


def _sc_agg_wide(y, src2d, dst2d, zeros128, zeros16, ones16):
    mesh = plsc.VectorSubcoreMesh(core_axis_name="c", subcore_axis_name="s")
    NBUF = 2
    K = NBUF - 1
    GROUPS = FULL_A // NBUF

    @functools.partial(
        pl.kernel,
        out_type=[
            jax.ShapeDtypeStruct((2, N, D), jnp.float32),
            jax.ShapeDtypeStruct((2, N, 16), jnp.float32),
        ],
        mesh=mesh,
        scratch_types=[
            [pltpu.VMEM((CA,), jnp.int32)] * NBUF,
            [pltpu.VMEM((CA,), jnp.int32)] * NBUF,
            [pltpu.VMEM((CA, D), jnp.float32)] * NBUF,
            pltpu.VMEM((CA, 16), jnp.float32),
            pltpu.VMEM_SHARED((N, D), jnp.float32),
            pltpu.VMEM_SHARED((N, 16), jnp.float32),
            [pltpu.SemaphoreType.DMA] * NBUF,
            [pltpu.SemaphoreType.DMA] * NBUF,
            [pltpu.SemaphoreType.DMA] * NBUF,
            [pltpu.SemaphoreType.DMA] * NBUF,
        ],
        compiler_params=pltpu.CompilerParams(use_tc_tiling_on_sc=False),
    )
    def k(y_hbm, src_hbm, dst_hbm, z128_hbm, z16_hbm, ones_hbm, feat_hbm, deg_hbm,
          srcs, dsts, rows, ones_v, acc_y, acc_d, isem, gsem, ssem, osem):
        c = lax.axis_index("c")
        s = lax.axis_index("s")
        w = s * 2 + c
        r0 = pl.multiple_of(s * RPS, 8)

        pltpu.sync_copy(z128_hbm.at[pl.ds(r0, RPS)], acc_y.at[pl.ds(r0, RPS)])
        pltpu.sync_copy(z16_hbm.at[pl.ds(r0, RPS)], acc_d.at[pl.ds(r0, RPS)])

        @pl.when(s == 15)
        def _():
            pltpu.sync_copy(z128_hbm.at[pl.ds(TAIL0, TAIL)],
                            acc_y.at[pl.ds(TAIL0, TAIL)])
            pltpu.sync_copy(z16_hbm.at[pl.ds(TAIL0, TAIL)],
                            acc_d.at[pl.ds(TAIL0, TAIL)])

        pltpu.sync_copy(ones_hbm, ones_v)
        plsc.subcore_barrier()

        c0 = w * FULL_A

        for j in range(K):
            pltpu.sync_copy(src_hbm.at[c0 + j], srcs[j])
            pltpu.sync_copy(dst_hbm.at[c0 + j], dsts[j])
            pltpu.async_copy(y_hbm.at[srcs[j]], rows[j], gsem[j])
        pltpu.async_copy(src_hbm.at[c0 + K], srcs[K], isem[K])
        pltpu.async_copy(dst_hbm.at[c0 + K], dsts[K], isem[K])

        def group(g, carry):
            for b in range(NBUF):
                i = g * NBUF + b
                bj = (b + K) % NBUF

                @pl.when(i + K < FULL_A)
                def _():
                    pltpu.make_async_copy(src_hbm.at[0], srcs[bj],
                                          isem[bj]).wait()
                    pltpu.make_async_copy(dst_hbm.at[0], dsts[bj],
                                          isem[bj]).wait()
                    pltpu.async_copy(y_hbm.at[srcs[bj]], rows[bj], gsem[bj])

                pltpu.make_async_copy(y_hbm.at[srcs[b]], rows[b],
                                      gsem[b]).wait()
                pltpu.async_copy(rows[b], acc_y.at[dsts[b]], ssem[b],
                                 add=True)
                pltpu.async_copy(ones_v, acc_d.at[dsts[b]], osem[b],
                                 add=True)
                pltpu.make_async_copy(rows[b], acc_y.at[dsts[b]],
                                      ssem[b]).wait()
                pltpu.make_async_copy(ones_v, acc_d.at[dsts[b]],
                                      osem[b]).wait()

                @pl.when(i + NBUF < FULL_A)
                def _():
                    pltpu.async_copy(src_hbm.at[c0 + i + NBUF], srcs[b],
                                     isem[b])
                    pltpu.async_copy(dst_hbm.at[c0 + i + NBUF], dsts[b],
                                     isem[b])
            return carry

        lax.fori_loop(0, GROUPS, group, 0)

        @pl.when(w < EXTRA_A)
        def _():
            xc = NW * FULL_A + w
            pltpu.sync_copy(src_hbm.at[xc], srcs[0])
            pltpu.sync_copy(dst_hbm.at[xc], dsts[0])
            pltpu.async_copy(y_hbm.at[srcs[0]], rows[0], gsem[0]).wait()
            pltpu.sync_copy(rows[0], acc_y.at[dsts[0]], add=True)
            pltpu.sync_copy(ones_v, acc_d.at[dsts[0]], add=True)

        plsc.subcore_barrier()
        pltpu.sync_copy(acc_y.at[pl.ds(r0, RPS)], feat_hbm.at[c, pl.ds(r0, RPS)])
        pltpu.sync_copy(acc_d.at[pl.ds(r0, RPS)], deg_hbm.at[c, pl.ds(r0, RPS)])

        @pl.when(s == 15)
        def _():
            pltpu.sync_copy(acc_y.at[pl.ds(TAIL0, TAIL)],
                            feat_hbm.at[c, pl.ds(TAIL0, TAIL)])
            pltpu.sync_copy(acc_d.at[pl.ds(TAIL0, TAIL)],
                            deg_hbm.at[c, pl.ds(TAIL0, TAIL)])

    return k(y, src2d, dst2d, zeros128, zeros16, ones16)


def _sc_agg_narrow(zs, src2d, dst2d, zeros16):
    mesh = plsc.VectorSubcoreMesh(core_axis_name="c", subcore_axis_name="s")
    NBUF = 6
    GROUPS = FULL_B // NBUF

    @functools.partial(
        pl.kernel,
        out_type=jax.ShapeDtypeStruct((2, N, 16), jnp.float32),
        mesh=mesh,
        scratch_types=[
            pltpu.VMEM((FULL_B, CB), jnp.int32),
            pltpu.VMEM((FULL_B, CB), jnp.int32),
            pltpu.VMEM((CB,), jnp.int32),
            pltpu.VMEM((CB,), jnp.int32),
            [pltpu.VMEM((CB, 16), jnp.float32)] * NBUF,
            pltpu.VMEM_SHARED((N, 16), jnp.float32),
            [pltpu.SemaphoreType.DMA] * NBUF,
            [pltpu.SemaphoreType.DMA] * NBUF,
            pltpu.SemaphoreType.DMA,
        ],
        compiler_params=pltpu.CompilerParams(use_tc_tiling_on_sc=False),
    )
    def k(zs_hbm, src_hbm, dst_hbm, z16_hbm, out_hbm,
          src_v, dst_v, srcx_v, dstx_v, rows, acc, gsem, ssem, sem):
        c = lax.axis_index("c")
        s = lax.axis_index("s")
        w = s * 2 + c
        r0 = pl.multiple_of(s * RPS, 8)

        pltpu.sync_copy(z16_hbm.at[pl.ds(r0, RPS)], acc.at[pl.ds(r0, RPS)])

        @pl.when(s == 15)
        def _():
            pltpu.sync_copy(z16_hbm.at[pl.ds(TAIL0, TAIL)],
                            acc.at[pl.ds(TAIL0, TAIL)])

        c0 = w * FULL_B
        pltpu.sync_copy(src_hbm.at[pl.ds(c0, FULL_B)], src_v)
        pltpu.sync_copy(dst_hbm.at[pl.ds(c0, FULL_B)], dst_v)
        plsc.subcore_barrier()

        for b in range(NBUF):
            pltpu.async_copy(zs_hbm.at[src_v.at[b]], rows[b], gsem[b])

        def group(g, carry):
            for b in range(NBUF):
                i = g * NBUF + b
                pltpu.make_async_copy(zs_hbm.at[src_v.at[b]], rows[b],
                                      gsem[b]).wait()
                pltpu.async_copy(rows[b], acc.at[dst_v.at[i]], ssem[b],
                                 add=True)
                pltpu.make_async_copy(rows[b], acc.at[dst_v.at[i]],
                                      ssem[b]).wait()

                @pl.when(i + NBUF < FULL_B)
                def _():
                    pltpu.async_copy(zs_hbm.at[src_v.at[i + NBUF]], rows[b],
                                     gsem[b])
            return carry

        lax.fori_loop(0, GROUPS, group, 0)

        @pl.when(w < EXTRA_B)
        def _():
            xc = NW * FULL_B + w
            pltpu.sync_copy(src_hbm.at[xc], srcx_v)
            pltpu.sync_copy(dst_hbm.at[xc], dstx_v)
            pltpu.async_copy(zs_hbm.at[srcx_v], rows[0], sem).wait()
            pltpu.sync_copy(rows[0], acc.at[dstx_v], add=True)

        plsc.subcore_barrier()
        pltpu.sync_copy(acc.at[pl.ds(r0, RPS)], out_hbm.at[c, pl.ds(r0, RPS)])

        @pl.when(s == 15)
        def _():
            pltpu.sync_copy(acc.at[pl.ds(TAIL0, TAIL)],
                            out_hbm.at[c, pl.ds(TAIL0, TAIL)])

    return k(zs, src2d, dst2d, zeros16)


def kernel(x, edge_index, W_self1, W_neigh1, b1, W_self2, W_neigh2, b2):
    srcA = edge_index[0].reshape(CHUNKS_A, CA)
    dstA = edge_index[1].reshape(CHUNKS_A, CA)
    srcB = edge_index[0].reshape(CHUNKS_B, CB)
    dstB = edge_index[1].reshape(CHUNKS_B, CB)

    y, xs = _tc1(x, W_neigh1, W_self1, b1)

    zeros128 = jnp.zeros((N, D), jnp.float32)
    zeros16 = jnp.zeros((N, 16), jnp.float32)
    ones16 = jnp.ones((CA, 16), jnp.float32)

    feat, deg = _sc_agg_wide(y, srcA, dstA, zeros128, zeros16, ones16)

    W2 = jnp.concatenate([W_neigh2, W_self2], axis=1)
    b2v = jnp.concatenate([jnp.zeros((1,), jnp.float32), b2])[None, :]
    zs = _tc2(xs, feat[0], feat[1], deg[0], deg[1], W2, b2v)

    accB = _sc_agg_narrow(zs, srcB, dstB, zeros16)

    return _tc3(zs, accB[0], accB[1])
